# Initial kernel scaffold; baseline (speedup 1.0000x reference)
import jax
import jax.numpy as jnp
from jax.experimental import pallas as pl


def kernel(bbox, label, rois):
    raise NotImplementedError("write your pallas kernel here")


# speedup vs baseline: 380.1496x; 380.1496x over previous
import functools

import numpy as np
import jax
import jax.numpy as jnp
from jax import lax
from jax.experimental import pallas as pl
from jax.experimental.pallas import tpu as pltpu
from jax.experimental.pallas import tpu_sc as plsc

_N_ROIS = 20000
_N_GT = 64
_N = _N_ROIS + _N_GT
_NW = 32
_CHUNK = 640
_NPAD = _NW * _CHUNK
_RBLK = 512
_SBUF = 2048
_MARGIN = 80
_SLEN = 1 << 18

_STREAM_NP = np.random.RandomState(111).randint(
    0, 2**32, size=_SLEN, dtype=np.uint32).view(np.int32)


def _iou_tc_body(rois_ref, bbox_ref, max_ref, amax_ref):
    i = pl.program_id(0)
    rx1 = rois_ref[0:1, :]
    ry1 = rois_ref[1:2, :]
    rx2 = rois_ref[2:3, :]
    ry2 = rois_ref[3:4, :]
    gx1 = bbox_ref[:, 0:1]
    gy1 = bbox_ref[:, 1:2]
    gx2 = bbox_ref[:, 2:3]
    gy2 = bbox_ref[:, 3:4]
    ltx = jnp.maximum(rx1, gx1)
    lty = jnp.maximum(ry1, gy1)
    rbx = jnp.minimum(rx2, gx2)
    rby = jnp.minimum(ry2, gy2)
    wx = jnp.maximum(rbx - ltx, 0.0)
    wy = jnp.maximum(rby - lty, 0.0)
    inter = wx * wy
    area_r = (rx2 - rx1) * (ry2 - ry1)
    area_g = (gx2 - gx1) * (gy2 - gy1)
    iou = inter / (area_r + area_g - inter)
    m = jnp.max(iou, axis=0, keepdims=True)
    ids = lax.broadcasted_iota(jnp.int32, (_N_GT, _RBLK), 0)
    am = jnp.min(jnp.where(iou == m, ids, _N_GT), axis=0, keepdims=True)
    col = i * _RBLK + lax.broadcasted_iota(jnp.int32, (1, _RBLK), 1)
    valid = col < _N
    max_ref[...] = jnp.where(valid, m, -1.0)
    amax_ref[...] = jnp.where(valid, am, 0)


_iou_call = pl.pallas_call(
    _iou_tc_body,
    grid=(_NPAD // _RBLK,),
    in_specs=[
        pl.BlockSpec((4, _RBLK), lambda i: (0, i)),
        pl.BlockSpec((_N_GT, 4), lambda i: (0, 0)),
    ],
    out_specs=[
        pl.BlockSpec((1, _RBLK), lambda i: (0, i)),
        pl.BlockSpec((1, _RBLK), lambda i: (0, i)),
    ],
    out_shape=[
        jax.ShapeDtypeStruct((1, _NPAD), jnp.float32),
        jax.ShapeDtypeStruct((1, _NPAD), jnp.int32),
    ],
)


def _splat(x):
    return jnp.full((16,), x, jnp.int32)


def _sread1(ref, i):
    return plsc.load_gather(ref, [_splat(0) + i])[0]


def _sread2(ref, i, j):
    return plsc.load_gather(ref, [_splat(0) + i, _splat(0) + j])[0]


def _swrite1(ref, i, val):
    lane0 = lax.iota(jnp.int32, 16) == 0
    plsc.store_scatter(ref, [_splat(0) + i], _splat(0) + val, mask=lane0)


_mesh = plsc.VectorSubcoreMesh(core_axis_name="c", subcore_axis_name="s")


@functools.partial(
    pl.kernel,
    out_type=[
        jax.ShapeDtypeStruct((_NW, _CHUNK), jnp.int32),
        jax.ShapeDtypeStruct((_NW, _CHUNK), jnp.int32),
        jax.ShapeDtypeStruct((_NW, 16), jnp.int32),
    ],
    mesh=_mesh,
    compiler_params=pltpu.CompilerParams(
        needs_layout_passes=False, use_tc_tiling_on_sc=False),
    scratch_types=[
        pltpu.VMEM((_CHUNK,), jnp.float32),
        pltpu.VMEM((_CHUNK,), jnp.int32),
        pltpu.VMEM((_CHUNK,), jnp.int32),
        pltpu.VMEM((16,), jnp.int32),
    ],
)
def _compact_call(iou_hbm, pos_hbm, neg_hbm, cnt_hbm,
                  iou_v, pos_v, neg_v, cnt_v):
    wid = lax.axis_index("c") * 16 + lax.axis_index("s")
    base = wid * _CHUNK
    pltpu.sync_copy(iou_hbm.at[pl.ds(base, _CHUNK)], iou_v)
    iota = lax.iota(jnp.int32, 16)
    pos_cnt = jnp.int32(0)
    neg_cnt = jnp.int32(0)
    for t in range(_CHUNK // 16):
        v = iou_v[pl.ds(t * 16, 16)]
        gidx = base + t * 16 + iota
        pm = v >= 0.5
        nm = jnp.logical_and(v < 0.5, v >= 0.0)
        ppos = pos_cnt + plsc.cumsum(pm.astype(jnp.int32)) - 1
        plsc.store_scatter(pos_v, [ppos], gidx, mask=pm)
        pos_cnt = pos_cnt + jnp.sum(pm.astype(jnp.int32))
        npos = neg_cnt + plsc.cumsum(nm.astype(jnp.int32)) - 1
        plsc.store_scatter(neg_v, [npos], gidx, mask=nm)
        neg_cnt = neg_cnt + jnp.sum(nm.astype(jnp.int32))
    cnt_v[...] = jnp.where(iota == 0, pos_cnt,
                           jnp.where(iota == 1, neg_cnt, 0))
    pltpu.sync_copy(pos_v, pos_hbm.at[wid])
    pltpu.sync_copy(neg_v, neg_hbm.at[wid])
    pltpu.sync_copy(cnt_v, cnt_hbm.at[wid])


@functools.partial(
    pl.kernel,
    out_type=[
        jax.ShapeDtypeStruct((128,), jnp.int32),
        jax.ShapeDtypeStruct((4, 128), jnp.float32),
        jax.ShapeDtypeStruct((4, 128), jnp.float32),
    ],
    mesh=_mesh,
    compiler_params=pltpu.CompilerParams(
        needs_layout_passes=False, use_tc_tiling_on_sc=False),
    scratch_types=[
        pltpu.VMEM((_NW, 16), jnp.int32),
        pltpu.VMEM((_NW, _CHUNK), jnp.int32),
        pltpu.VMEM((_NW, _CHUNK), jnp.int32),
        pltpu.VMEM((_NPAD,), jnp.int32),
        pltpu.VMEM((_SBUF,), jnp.int32),
        pltpu.VMEM((_NPAD,), jnp.int32),
        pltpu.SMEM((_NW + 2,), jnp.int32),
        pltpu.SMEM((_NW + 2,), jnp.int32),
        pltpu.VMEM((128,), jnp.int32),
        pltpu.VMEM((128,), jnp.int32),
        pltpu.VMEM((_N_GT, 4), jnp.float32),
        pltpu.VMEM((_N_GT,), jnp.int32),
        pltpu.VMEM((128, 16), jnp.float32),
        pltpu.VMEM((4, 128), jnp.float32),
        pltpu.VMEM((4, 128), jnp.float32),
        pltpu.SemaphoreType.DMA,
    ],
)
def _sample_call(cnt_hbm, pos_hbm, neg_hbm, amax_hbm, stream_hbm, rois16_hbm,
                 bbox_hbm, label_hbm,
                 cls_hbm, mgt_hbm, srois_hbm,
                 cnt_v, posl_v, negl_v, amax_v, sbuf_v, perm_v, ppos_s,
                 pneg_s, keep_v, cls_v, bbox_v, label_v, roisg_v, mgt_t_v,
                 srois_t_v, sem):
    wid = lax.axis_index("c") * 16 + lax.axis_index("s")

    @pl.when(wid == 0)
    def _():
        pltpu.sync_copy(cnt_hbm, cnt_v)
        pltpu.sync_copy(pos_hbm, posl_v)
        pltpu.sync_copy(neg_hbm, negl_v)
        pltpu.sync_copy(amax_hbm, amax_v)
        pltpu.sync_copy(bbox_hbm, bbox_v)
        pltpu.sync_copy(label_hbm, label_v)
        pltpu.sync_copy(stream_hbm.at[pl.ds(0, _SBUF)], sbuf_v)

        iota = lax.iota(jnp.int32, 16)

        def pref_body(t, accs):
            ap, an = accs
            ppos_s[t] = ap
            pneg_s[t] = an
            return (ap + _sread2(cnt_v, t, 0), an + _sread2(cnt_v, t, 1))

        pop_pos, pop_neg = lax.fori_loop(
            0, _NW, pref_body, (jnp.int32(0), jnp.int32(0)))
        ppos_s[_NW] = pop_pos
        pneg_s[_NW] = pop_neg

        n_init = (jnp.maximum(pop_pos, 32) + 15) // 16

        def init_body(t, c):
            b = t * 16
            plsc.store_scatter(perm_v, [b + iota], b + iota)
            return c
        lax.fori_loop(0, n_init, init_body, 0)

        def smear(x):
            x = x | (x >> 1)
            x = x | (x >> 2)
            x = x | (x >> 4)
            x = x | (x >> 8)
            x = x | (x >> 16)
            return x

        def ensure(ptr, base):
            def do(_):
                nb = jnp.minimum((ptr // 8) * 8, _SLEN - _SBUF)
                nb = pl.multiple_of(nb, 8)
                pltpu.sync_copy(stream_hbm.at[pl.ds(nb, _SBUF)], sbuf_v)
                return nb
            need = jnp.logical_or(ptr < base, ptr + _MARGIN > base + _SBUF)
            return lax.cond(need, do, lambda _: base, 0)

        def draw(ptr, base, bound_u):
            mask = smear(bound_u)

            def cond(c):
                return c[0] > bound_u

            def body(c):
                _, p = c
                off = jnp.minimum(p - base, _SBUF - 1)
                w = _sread1(sbuf_v, off).astype(jnp.uint32)
                return (w & mask, p + 1)

            v, ptr = lax.while_loop(cond, body, (bound_u + jnp.uint32(1), ptr))
            return v, ptr

        lane01 = iota < 2

        def fy_body(d, carry):
            ptr, base = carry
            i = pop_pos - 1 - d
            base = ensure(ptr, base)
            j_u, ptr = draw(ptr, base, i.astype(jnp.uint32))
            j = j_u.astype(jnp.int32)
            ij = jnp.where(iota == 0, i, j)
            pij = plsc.load_gather(perm_v, [ij])
            pi = pij[0]
            pj = pij[1]
            plsc.store_scatter(perm_v, [ij],
                               jnp.where(iota == 0, pj, pi), mask=lane01)
            return (ptr, base)

        _, base_end = lax.fori_loop(
            0, jnp.maximum(pop_pos - 1, 0), fy_body,
            (jnp.int32(0), jnp.int32(0)))

        def lookup(list_ref, pref_ref, r):
            def b(w, acc):
                return acc + jnp.where(pref_ref[w + 1] <= r, 1, 0)
            w = lax.fori_loop(0, _NW, b, jnp.int32(0))
            w = jnp.minimum(w, _NW - 1)
            off = jnp.clip(r - pref_ref[w], 0, _CHUNK - 1)
            return _sread2(list_ref, w, off)

        def keep_pos_body(k, c):
            r = _sread1(perm_v, k)
            idx = lax.cond(
                r < pop_pos,
                lambda rr: lookup(posl_v, ppos_s, rr),
                lambda rr: lookup(negl_v, pneg_s, rr - pop_pos),
                r)
            _swrite1(keep_v, k, idx)
            return c
        lax.fori_loop(0, 32, keep_pos_body, 0)

        bound_n = (pop_neg - 1).astype(jnp.uint32)

        def neg_body(k, carry):
            ptr, base = carry
            base = ensure(ptr, base)

            def do(p):
                return draw(p, base, bound_n)

            def skip(p):
                return jnp.uint32(0), p

            v, ptr = lax.cond(pop_neg > 1, do, skip, ptr)
            d = v.astype(jnp.int32)
            idx = jnp.where(pop_neg == 0, jnp.int32(0),
                            lookup(negl_v, pneg_s, d))
            _swrite1(keep_v, 32 + k, idx)
            return (ptr, base)

        lax.fori_loop(0, 96, neg_body, (jnp.int32(0), base_end))

        for c in range(8):
            kvec = keep_v[pl.ds(c * 16, 16)]
            avec = plsc.load_gather(amax_v, [kvec])
            lvec = plsc.load_gather(label_v, [avec])
            if c < 2:
                cls_v[pl.ds(c * 16, 16)] = lvec + 1
            else:
                cls_v[pl.ds(c * 16, 16)] = jnp.zeros((16,), jnp.int32)
            for col in range(4):
                g = plsc.load_gather(bbox_v, [avec, _splat(col)])
                mgt_t_v[col, pl.ds(c * 16, 16)] = g

        pltpu.async_copy(rois16_hbm.at[keep_v], roisg_v, sem).wait()
        for c in range(8):
            rows = c * 16 + iota
            for col in range(4):
                g = plsc.load_gather(roisg_v, [rows, _splat(col)])
                srois_t_v[col, pl.ds(c * 16, 16)] = g

        pltpu.sync_copy(cls_v, cls_hbm)
        pltpu.sync_copy(mgt_t_v, mgt_hbm)
        pltpu.sync_copy(srois_t_v, srois_hbm)


def _encode_tc_body(g_ref, p_ref, out_ref):
    g = g_ref[...]
    p = p_ref[...]
    gcx = (g[2:3] + g[0:1]) / 2.0
    gcy = (g[3:4] + g[1:2]) / 2.0
    gw = g[2:3] - g[0:1]
    gh = g[3:4] - g[1:2]
    pcx = (p[2:3] + p[0:1]) / 2.0
    pcy = (p[3:4] + p[1:2]) / 2.0
    pw = p[2:3] - p[0:1]
    ph = p[3:4] - p[1:2]
    tx = (gcx - pcx) / pw
    ty = (gcy - pcy) / ph
    tw = jnp.log(gw / pw)
    th = jnp.log(gh / ph)
    out_ref[...] = jnp.concatenate([tx, ty, tw, th], axis=0)


_encode_call = pl.pallas_call(
    _encode_tc_body,
    out_shape=jax.ShapeDtypeStruct((4, 128), jnp.float32),
)


def kernel(bbox, label, rois):
    bbox0 = bbox[0]
    label0 = label[0]
    rois_cat = jnp.concatenate([rois, bbox0], axis=0)
    rois_t = jnp.pad(rois_cat, ((0, _NPAD - _N), (0, 0))).T
    iou_max2, amax2 = _iou_call(rois_t, bbox0)
    iou_max = iou_max2.reshape(_NPAD)
    amax = amax2.reshape(_NPAD)
    pos_l, neg_l, cnts = _compact_call(iou_max)
    rois16 = jnp.pad(rois_cat, ((0, 0), (0, 12)))
    stream = jnp.asarray(_STREAM_NP)
    cls, mgt_t, srois_t = _sample_call(
        cnts, pos_l, neg_l, amax, stream, rois16, bbox0, label0)
    sample_rois = srois_t.T
    reg = _encode_call(mgt_t, srois_t).T
    return (cls, reg, sample_rois)


# speedup vs baseline: 448.1242x; 1.1788x over previous
import functools

import numpy as np
import jax
import jax.numpy as jnp
from jax import lax
from jax.experimental import pallas as pl
from jax.experimental.pallas import tpu as pltpu
from jax.experimental.pallas import tpu_sc as plsc

_N_ROIS = 20000
_N_GT = 64
_N = _N_ROIS + _N_GT
_NW = 32
_CHUNK = 640
_NPAD = _NW * _CHUNK
_RBLK = 512
_SBUF = 2048
_MARGIN = 80
_SLEN = 1 << 18

_STREAM_NP = np.random.RandomState(111).randint(
    0, 2**32, size=_SLEN, dtype=np.uint32).view(np.int32)


def _iou_tc_body(rois_ref, bbox_ref, max_ref, amax_ref):
    i = pl.program_id(0)
    rx1 = rois_ref[0:1, :]
    ry1 = rois_ref[1:2, :]
    rx2 = rois_ref[2:3, :]
    ry2 = rois_ref[3:4, :]
    gx1 = bbox_ref[:, 0:1]
    gy1 = bbox_ref[:, 1:2]
    gx2 = bbox_ref[:, 2:3]
    gy2 = bbox_ref[:, 3:4]
    ltx = jnp.maximum(rx1, gx1)
    lty = jnp.maximum(ry1, gy1)
    rbx = jnp.minimum(rx2, gx2)
    rby = jnp.minimum(ry2, gy2)
    wx = jnp.maximum(rbx - ltx, 0.0)
    wy = jnp.maximum(rby - lty, 0.0)
    inter = wx * wy
    area_r = (rx2 - rx1) * (ry2 - ry1)
    area_g = (gx2 - gx1) * (gy2 - gy1)
    iou = inter / (area_r + area_g - inter)
    m = jnp.max(iou, axis=0, keepdims=True)
    ids = lax.broadcasted_iota(jnp.int32, (_N_GT, _RBLK), 0)
    am = jnp.min(jnp.where(iou == m, ids, _N_GT), axis=0, keepdims=True)
    col = i * _RBLK + lax.broadcasted_iota(jnp.int32, (1, _RBLK), 1)
    valid = col < _N
    max_ref[...] = jnp.where(valid, m, -1.0)
    amax_ref[...] = jnp.where(valid, am, 0)


_iou_call = pl.pallas_call(
    _iou_tc_body,
    grid=(_NPAD // _RBLK,),
    in_specs=[
        pl.BlockSpec((4, _RBLK), lambda i: (0, i)),
        pl.BlockSpec((_N_GT, 4), lambda i: (0, 0)),
    ],
    out_specs=[
        pl.BlockSpec((1, _RBLK), lambda i: (0, i)),
        pl.BlockSpec((1, _RBLK), lambda i: (0, i)),
    ],
    out_shape=[
        jax.ShapeDtypeStruct((1, _NPAD), jnp.float32),
        jax.ShapeDtypeStruct((1, _NPAD), jnp.int32),
    ],
)


def _splat(x):
    return jnp.full((16,), x, jnp.int32)


def _sread1(ref, i):
    return plsc.load_gather(ref, [_splat(0) + i])[0]


def _sread2(ref, i, j):
    return plsc.load_gather(ref, [_splat(0) + i, _splat(0) + j])[0]


def _swrite1(ref, i, val):
    lane0 = lax.iota(jnp.int32, 16) == 0
    plsc.store_scatter(ref, [_splat(0) + i], _splat(0) + val, mask=lane0)


_mesh = plsc.VectorSubcoreMesh(core_axis_name="c", subcore_axis_name="s")


@functools.partial(
    pl.kernel,
    out_type=[
        jax.ShapeDtypeStruct((_NW, _CHUNK), jnp.int32),
        jax.ShapeDtypeStruct((_NW, _CHUNK), jnp.int32),
        jax.ShapeDtypeStruct((_NW, 16), jnp.int32),
    ],
    mesh=_mesh,
    compiler_params=pltpu.CompilerParams(
        needs_layout_passes=False, use_tc_tiling_on_sc=False),
    scratch_types=[
        pltpu.VMEM((_CHUNK,), jnp.float32),
        pltpu.VMEM((_CHUNK,), jnp.int32),
        pltpu.VMEM((_CHUNK,), jnp.int32),
        pltpu.VMEM((16,), jnp.int32),
    ],
)
def _compact_call(iou_hbm, pos_hbm, neg_hbm, cnt_hbm,
                  iou_v, pos_v, neg_v, cnt_v):
    wid = lax.axis_index("c") * 16 + lax.axis_index("s")
    base = wid * _CHUNK
    pltpu.sync_copy(iou_hbm.at[pl.ds(base, _CHUNK)], iou_v)
    iota = lax.iota(jnp.int32, 16)
    pos_cnt = jnp.int32(0)
    neg_cnt = jnp.int32(0)
    for t in range(_CHUNK // 16):
        v = iou_v[pl.ds(t * 16, 16)]
        gidx = base + t * 16 + iota
        pm = v >= 0.5
        nm = jnp.logical_and(v < 0.5, v >= 0.0)
        ppos = pos_cnt + plsc.cumsum(pm.astype(jnp.int32)) - 1
        plsc.store_scatter(pos_v, [ppos], gidx, mask=pm)
        pos_cnt = pos_cnt + jnp.sum(pm.astype(jnp.int32))
        npos = neg_cnt + plsc.cumsum(nm.astype(jnp.int32)) - 1
        plsc.store_scatter(neg_v, [npos], gidx, mask=nm)
        neg_cnt = neg_cnt + jnp.sum(nm.astype(jnp.int32))
    cnt_v[...] = jnp.where(iota == 0, pos_cnt,
                           jnp.where(iota == 1, neg_cnt, 0))
    pltpu.sync_copy(pos_v, pos_hbm.at[wid])
    pltpu.sync_copy(neg_v, neg_hbm.at[wid])
    pltpu.sync_copy(cnt_v, cnt_hbm.at[wid])


@functools.partial(
    pl.kernel,
    out_type=[
        jax.ShapeDtypeStruct((128,), jnp.int32),
        jax.ShapeDtypeStruct((4, 128), jnp.float32),
        jax.ShapeDtypeStruct((4, 128), jnp.float32),
    ],
    mesh=_mesh,
    compiler_params=pltpu.CompilerParams(
        needs_layout_passes=False, use_tc_tiling_on_sc=False),
    scratch_types=[
        pltpu.VMEM((_NW, 16), jnp.int32),
        pltpu.VMEM((_NW, _CHUNK), jnp.int32),
        pltpu.VMEM((_NW, _CHUNK), jnp.int32),
        pltpu.VMEM((_NPAD,), jnp.int32),
        pltpu.VMEM((_SBUF,), jnp.int32),
        pltpu.VMEM((_NPAD,), jnp.int32),
        pltpu.VMEM((_NW,), jnp.int32),
        pltpu.VMEM((_NW,), jnp.int32),
        pltpu.VMEM((_NW,), jnp.int32),
        pltpu.VMEM((_NW,), jnp.int32),
        pltpu.VMEM((128,), jnp.int32),
        pltpu.VMEM((128,), jnp.int32),
        pltpu.VMEM((_N_GT, 4), jnp.float32),
        pltpu.VMEM((_N_GT,), jnp.int32),
        pltpu.VMEM((128, 16), jnp.float32),
        pltpu.VMEM((4, 128), jnp.float32),
        pltpu.VMEM((4, 128), jnp.float32),
        pltpu.SemaphoreType.DMA,
    ],
)
def _sample_call(cnt_hbm, pos_hbm, neg_hbm, amax_hbm, stream_hbm, rois16_hbm,
                 bbox_hbm, label_hbm,
                 cls_hbm, mgt_hbm, srois_hbm,
                 cnt_v, posl_v, negl_v, amax_v, sbuf_v, perm_v, ipos_v,
                 epos_v, ineg_v, eneg_v, keep_v, cls_v, bbox_v, label_v,
                 roisg_v, mgt_t_v, srois_t_v, sem):
    wid = lax.axis_index("c") * 16 + lax.axis_index("s")

    @pl.when(wid == 0)
    def _():
        pltpu.sync_copy(cnt_hbm, cnt_v)
        pltpu.sync_copy(pos_hbm, posl_v)
        pltpu.sync_copy(neg_hbm, negl_v)
        pltpu.sync_copy(amax_hbm, amax_v)
        pltpu.sync_copy(bbox_hbm, bbox_v)
        pltpu.sync_copy(label_hbm, label_v)
        pltpu.sync_copy(stream_hbm.at[pl.ds(0, _SBUF)], sbuf_v)

        iota = lax.iota(jnp.int32, 16)

        def prefixes(col, i_ref, e_ref):
            c0 = plsc.load_gather(cnt_v, [iota, _splat(col)])
            c1 = plsc.load_gather(cnt_v, [iota + 16, _splat(col)])
            i0 = plsc.cumsum(c0)
            i1 = plsc.cumsum(c1) + i0[15]
            i_ref[pl.ds(0, 16)] = i0
            i_ref[pl.ds(16, 16)] = i1
            e_ref[pl.ds(0, 16)] = i0 - c0
            e_ref[pl.ds(16, 16)] = i1 - c1
            return i1[15]

        pop_pos = prefixes(0, ipos_v, epos_v)
        pop_neg = prefixes(1, ineg_v, eneg_v)

        n_init = (jnp.maximum(pop_pos, 32) + 15) // 16

        def init_body(t, c):
            b = t * 16
            plsc.store_scatter(perm_v, [b + iota], b + iota)
            return c
        lax.fori_loop(0, n_init, init_body, 0)

        def smear(x):
            x = x | (x >> 1)
            x = x | (x >> 2)
            x = x | (x >> 4)
            x = x | (x >> 8)
            x = x | (x >> 16)
            return x

        def ensure(ptr, base):
            def do(_):
                nb = jnp.minimum((ptr // 8) * 8, _SLEN - _SBUF)
                nb = pl.multiple_of(nb, 8)
                pltpu.sync_copy(stream_hbm.at[pl.ds(nb, _SBUF)], sbuf_v)
                return nb
            need = jnp.logical_or(ptr < base, ptr + _MARGIN > base + _SBUF)
            return lax.cond(need, do, lambda _: base, 0)

        def draw(ptr, base, bound_u):
            mask = smear(bound_u)

            def cond(c):
                return c[0] > bound_u

            def body(c):
                _, p = c
                off = jnp.minimum(p - base, _SBUF - 1)
                w = _sread1(sbuf_v, off).astype(jnp.uint32)
                return (w & mask, p + 1)

            v, ptr = lax.while_loop(cond, body, (bound_u + jnp.uint32(1), ptr))
            return v, ptr

        lane01 = iota < 2

        def fy_body(d, carry):
            ptr, base = carry
            i = pop_pos - 1 - d
            base = ensure(ptr, base)
            j_u, ptr = draw(ptr, base, i.astype(jnp.uint32))
            j = j_u.astype(jnp.int32)
            ij = jnp.where(iota == 0, i, j)
            pij = plsc.load_gather(perm_v, [ij])
            pi = pij[0]
            pj = pij[1]
            plsc.store_scatter(perm_v, [ij],
                               jnp.where(iota == 0, pj, pi), mask=lane01)
            return (ptr, base)

        _, base_end = lax.fori_loop(
            0, jnp.maximum(pop_pos - 1, 0), fy_body,
            (jnp.int32(0), jnp.int32(0)))

        def vlookup(list_ref, i_ref, e_ref, r):
            w = jnp.zeros((16,), jnp.int32)
            for step in (16, 8, 4, 2, 1):
                probe = jnp.minimum(w + (step - 1), _NW - 1)
                ipv = plsc.load_gather(i_ref, [probe])
                w = jnp.where(ipv <= r, w + step, w)
            w = jnp.minimum(w, _NW - 1)
            off = jnp.clip(r - plsc.load_gather(e_ref, [w]), 0, _CHUNK - 1)
            return plsc.load_gather(list_ref, [w, off])

        for c in range(2):
            r = perm_v[pl.ds(c * 16, 16)]
            pidx = vlookup(posl_v, ipos_v, epos_v, r)
            nidx = vlookup(negl_v, ineg_v, eneg_v, r - pop_pos)
            keep_v[pl.ds(c * 16, 16)] = jnp.where(r < pop_pos, pidx, nidx)

        bound_n = (pop_neg - 1).astype(jnp.uint32)

        def neg_body(k, carry):
            ptr, base = carry
            base = ensure(ptr, base)

            def do(p):
                return draw(p, base, bound_n)

            def skip(p):
                return jnp.uint32(0), p

            v, ptr = lax.cond(pop_neg > 1, do, skip, ptr)
            _swrite1(keep_v, 32 + k, v.astype(jnp.int32))
            return (ptr, base)

        lax.fori_loop(0, 96, neg_body, (jnp.int32(0), base_end))

        for c in range(2, 8):
            d = keep_v[pl.ds(c * 16, 16)]
            idx = vlookup(negl_v, ineg_v, eneg_v, d)
            keep_v[pl.ds(c * 16, 16)] = jnp.where(
                _splat(pop_neg) == 0, 0, idx)

        for c in range(8):
            kvec = keep_v[pl.ds(c * 16, 16)]
            avec = plsc.load_gather(amax_v, [kvec])
            lvec = plsc.load_gather(label_v, [avec])
            if c < 2:
                cls_v[pl.ds(c * 16, 16)] = lvec + 1
            else:
                cls_v[pl.ds(c * 16, 16)] = jnp.zeros((16,), jnp.int32)
            for col in range(4):
                g = plsc.load_gather(bbox_v, [avec, _splat(col)])
                mgt_t_v[col, pl.ds(c * 16, 16)] = g

        pltpu.async_copy(rois16_hbm.at[keep_v], roisg_v, sem).wait()
        for c in range(8):
            rows = c * 16 + iota
            for col in range(4):
                g = plsc.load_gather(roisg_v, [rows, _splat(col)])
                srois_t_v[col, pl.ds(c * 16, 16)] = g

        pltpu.sync_copy(cls_v, cls_hbm)
        pltpu.sync_copy(mgt_t_v, mgt_hbm)
        pltpu.sync_copy(srois_t_v, srois_hbm)


def _encode_tc_body(g_ref, p_ref, out_ref):
    g = g_ref[...]
    p = p_ref[...]
    gcx = (g[2:3] + g[0:1]) / 2.0
    gcy = (g[3:4] + g[1:2]) / 2.0
    gw = g[2:3] - g[0:1]
    gh = g[3:4] - g[1:2]
    pcx = (p[2:3] + p[0:1]) / 2.0
    pcy = (p[3:4] + p[1:2]) / 2.0
    pw = p[2:3] - p[0:1]
    ph = p[3:4] - p[1:2]
    tx = (gcx - pcx) / pw
    ty = (gcy - pcy) / ph
    tw = jnp.log(gw / pw)
    th = jnp.log(gh / ph)
    out_ref[...] = jnp.concatenate([tx, ty, tw, th], axis=0)


_encode_call = pl.pallas_call(
    _encode_tc_body,
    out_shape=jax.ShapeDtypeStruct((4, 128), jnp.float32),
)


def kernel(bbox, label, rois):
    bbox0 = bbox[0]
    label0 = label[0]
    rois_cat = jnp.concatenate([rois, bbox0], axis=0)
    rois_t = jnp.pad(rois_cat, ((0, _NPAD - _N), (0, 0))).T
    iou_max2, amax2 = _iou_call(rois_t, bbox0)
    iou_max = iou_max2.reshape(_NPAD)
    amax = amax2.reshape(_NPAD)
    pos_l, neg_l, cnts = _compact_call(iou_max)
    rois16 = jnp.pad(rois_cat, ((0, 0), (0, 12)))
    stream = jnp.asarray(_STREAM_NP)
    cls, mgt_t, srois_t = _sample_call(
        cnts, pos_l, neg_l, amax, stream, rois16, bbox0, label0)
    sample_rois = srois_t.T
    reg = _encode_call(mgt_t, srois_t).T
    return (cls, reg, sample_rois)


# speedup vs baseline: 465.7273x; 1.0393x over previous
import functools

import numpy as np
import jax
import jax.numpy as jnp
from jax import lax
from jax.experimental import pallas as pl
from jax.experimental.pallas import tpu as pltpu
from jax.experimental.pallas import tpu_sc as plsc

_N_ROIS = 20000
_N_GT = 64
_N = _N_ROIS + _N_GT
_NW = 32
_CHUNK = 640
_NPAD = _NW * _CHUNK
_RBLK = 512
_SBUF = 2048
_MARGIN = 80
_SLEN = 1 << 18

_STREAM_NP = np.random.RandomState(111).randint(
    0, 2**32, size=_SLEN, dtype=np.uint32).view(np.int32)


def _iou_tc_body(rois_ref, bbox_ref, max_ref, amax_ref):
    i = pl.program_id(0)
    rx1 = rois_ref[0:1, :]
    ry1 = rois_ref[1:2, :]
    rx2 = rois_ref[2:3, :]
    ry2 = rois_ref[3:4, :]
    gx1 = bbox_ref[:, 0:1]
    gy1 = bbox_ref[:, 1:2]
    gx2 = bbox_ref[:, 2:3]
    gy2 = bbox_ref[:, 3:4]
    ltx = jnp.maximum(rx1, gx1)
    lty = jnp.maximum(ry1, gy1)
    rbx = jnp.minimum(rx2, gx2)
    rby = jnp.minimum(ry2, gy2)
    wx = jnp.maximum(rbx - ltx, 0.0)
    wy = jnp.maximum(rby - lty, 0.0)
    inter = wx * wy
    area_r = (rx2 - rx1) * (ry2 - ry1)
    area_g = (gx2 - gx1) * (gy2 - gy1)
    iou = inter / (area_r + area_g - inter)
    m = jnp.max(iou, axis=0, keepdims=True)
    ids = lax.broadcasted_iota(jnp.int32, (_N_GT, _RBLK), 0)
    am = jnp.min(jnp.where(iou == m, ids, _N_GT), axis=0, keepdims=True)
    col = i * _RBLK + lax.broadcasted_iota(jnp.int32, (1, _RBLK), 1)
    valid = col < _N
    max_ref[...] = jnp.where(valid, m, -1.0)
    amax_ref[...] = jnp.where(valid, am, 0)


_iou_call = pl.pallas_call(
    _iou_tc_body,
    grid=(_NPAD // _RBLK,),
    in_specs=[
        pl.BlockSpec((4, _RBLK), lambda i: (0, i)),
        pl.BlockSpec((_N_GT, 4), lambda i: (0, 0)),
    ],
    out_specs=[
        pl.BlockSpec((1, _RBLK), lambda i: (0, i)),
        pl.BlockSpec((1, _RBLK), lambda i: (0, i)),
    ],
    out_shape=[
        jax.ShapeDtypeStruct((1, _NPAD), jnp.float32),
        jax.ShapeDtypeStruct((1, _NPAD), jnp.int32),
    ],
)


def _splat(x):
    return jnp.full((16,), x, jnp.int32)


def _sread1(ref, i):
    return plsc.load_gather(ref, [_splat(0) + i])[0]


def _sread2(ref, i, j):
    return plsc.load_gather(ref, [_splat(0) + i, _splat(0) + j])[0]


def _swrite1(ref, i, val):
    lane0 = lax.iota(jnp.int32, 16) == 0
    plsc.store_scatter(ref, [_splat(0) + i], _splat(0) + val, mask=lane0)


_mesh = plsc.VectorSubcoreMesh(core_axis_name="c", subcore_axis_name="s")
_NSUB = 16
_CHUNK2 = _NPAD // _NSUB


@functools.partial(
    pl.kernel,
    out_type=[
        jax.ShapeDtypeStruct((128,), jnp.int32),
        jax.ShapeDtypeStruct((4, 128), jnp.float32),
        jax.ShapeDtypeStruct((4, 128), jnp.float32),
    ],
    mesh=_mesh,
    compiler_params=pltpu.CompilerParams(
        needs_layout_passes=False, use_tc_tiling_on_sc=False),
    scratch_types=[
        pltpu.VMEM((_CHUNK2,), jnp.float32),
        pltpu.VMEM((_CHUNK2,), jnp.int32),
        pltpu.VMEM((_CHUNK2,), jnp.int32),
        pltpu.VMEM((16,), jnp.int32),
        pltpu.VMEM_SHARED((_NSUB, _CHUNK2), jnp.int32),
        pltpu.VMEM_SHARED((_NSUB, _CHUNK2), jnp.int32),
        pltpu.VMEM_SHARED((_NSUB, 16), jnp.int32),
        pltpu.VMEM((_NSUB, 16), jnp.int32),
        pltpu.VMEM((_NSUB, _CHUNK2), jnp.int32),
        pltpu.VMEM((_NSUB, _CHUNK2), jnp.int32),
        pltpu.VMEM((_NPAD,), jnp.int32),
        pltpu.VMEM((_SBUF,), jnp.int32),
        pltpu.VMEM((_NPAD,), jnp.int32),
        pltpu.VMEM((_NSUB,), jnp.int32),
        pltpu.VMEM((_NSUB,), jnp.int32),
        pltpu.VMEM((_NSUB,), jnp.int32),
        pltpu.VMEM((_NSUB,), jnp.int32),
        pltpu.VMEM((128,), jnp.int32),
        pltpu.VMEM((128,), jnp.int32),
        pltpu.VMEM((_N_GT, 4), jnp.float32),
        pltpu.VMEM((_N_GT,), jnp.int32),
        pltpu.VMEM((128, 16), jnp.float32),
        pltpu.VMEM((4, 128), jnp.float32),
        pltpu.VMEM((4, 128), jnp.float32),
        pltpu.SemaphoreType.DMA,
    ],
)
def _fused_call(iou_hbm, amax_hbm, stream_hbm, rois16_hbm, bbox_hbm,
                label_hbm,
                cls_hbm, mgt_hbm, srois_hbm,
                iou_v, posloc_v, negloc_v, cnt16_v, pos_sh, neg_sh, cnt_sh,
                cnt_v, posl_v, negl_v, amax_v, sbuf_v, perm_v, ipos_v,
                epos_v, ineg_v, eneg_v, keep_v, cls_v, bbox_v, label_v,
                roisg_v, mgt_t_v, srois_t_v, sem):
    core = lax.axis_index("c")
    sub = lax.axis_index("s")

    @pl.when(core == 0)
    def _():
        base = sub * _CHUNK2
        pltpu.sync_copy(iou_hbm.at[pl.ds(base, _CHUNK2)], iou_v)
        iota = lax.iota(jnp.int32, 16)
        pos_cnt = jnp.int32(0)
        neg_cnt = jnp.int32(0)
        for t in range(_CHUNK2 // 16):
            v = iou_v[pl.ds(t * 16, 16)]
            gidx = base + t * 16 + iota
            pm = v >= 0.5
            nm = jnp.logical_and(v < 0.5, v >= 0.0)
            ppos = pos_cnt + plsc.cumsum(pm.astype(jnp.int32)) - 1
            plsc.store_scatter(posloc_v, [ppos], gidx, mask=pm)
            pos_cnt = pos_cnt + jnp.sum(pm.astype(jnp.int32))
            npos = neg_cnt + plsc.cumsum(nm.astype(jnp.int32)) - 1
            plsc.store_scatter(negloc_v, [npos], gidx, mask=nm)
            neg_cnt = neg_cnt + jnp.sum(nm.astype(jnp.int32))
        cnt16_v[...] = jnp.where(iota == 0, pos_cnt,
                                 jnp.where(iota == 1, neg_cnt, 0))
        pltpu.sync_copy(posloc_v, pos_sh.at[sub])
        pltpu.sync_copy(negloc_v, neg_sh.at[sub])
        pltpu.sync_copy(cnt16_v, cnt_sh.at[sub])

    plsc.subcore_barrier()

    @pl.when(jnp.logical_and(core == 0, sub == 0))
    def _():
        pltpu.sync_copy(cnt_sh, cnt_v)
        pltpu.sync_copy(pos_sh, posl_v)
        pltpu.sync_copy(neg_sh, negl_v)
        pltpu.sync_copy(amax_hbm, amax_v)
        pltpu.sync_copy(bbox_hbm, bbox_v)
        pltpu.sync_copy(label_hbm, label_v)
        pltpu.sync_copy(stream_hbm.at[pl.ds(0, _SBUF)], sbuf_v)

        iota = lax.iota(jnp.int32, 16)

        def prefixes(col, i_ref, e_ref):
            c0 = plsc.load_gather(cnt_v, [iota, _splat(col)])
            i0 = plsc.cumsum(c0)
            i_ref[...] = i0
            e_ref[...] = i0 - c0
            return i0[15]

        pop_pos = prefixes(0, ipos_v, epos_v)
        pop_neg = prefixes(1, ineg_v, eneg_v)

        n_init = (jnp.maximum(pop_pos, 32) + 15) // 16

        def init_body(t, c):
            b = t * 16
            plsc.store_scatter(perm_v, [b + iota], b + iota)
            return c
        lax.fori_loop(0, n_init, init_body, 0)

        def smear(x):
            x = x | (x >> 1)
            x = x | (x >> 2)
            x = x | (x >> 4)
            x = x | (x >> 8)
            x = x | (x >> 16)
            return x

        def ensure(ptr, base):
            def do(_):
                nb = jnp.minimum((ptr // 8) * 8, _SLEN - _SBUF)
                nb = pl.multiple_of(nb, 8)
                pltpu.sync_copy(stream_hbm.at[pl.ds(nb, _SBUF)], sbuf_v)
                return nb
            need = jnp.logical_or(ptr < base, ptr + _MARGIN > base + _SBUF)
            return lax.cond(need, do, lambda _: base, 0)

        def draw(ptr, base, bound_u):
            mask = smear(bound_u)

            def cond(c):
                return c[0] > bound_u

            def body(c):
                _, p = c
                off = jnp.minimum(p - base, _SBUF - 1)
                w = _sread1(sbuf_v, off).astype(jnp.uint32)
                return (w & mask, p + 1)

            v, ptr = lax.while_loop(cond, body, (bound_u + jnp.uint32(1), ptr))
            return v, ptr

        lane01 = iota < 2

        def fy_body(d, carry):
            ptr, base = carry
            i = pop_pos - 1 - d
            base = ensure(ptr, base)
            j_u, ptr = draw(ptr, base, i.astype(jnp.uint32))
            j = j_u.astype(jnp.int32)
            ij = jnp.where(iota == 0, i, j)
            pij = plsc.load_gather(perm_v, [ij])
            pi = pij[0]
            pj = pij[1]
            plsc.store_scatter(perm_v, [ij],
                               jnp.where(iota == 0, pj, pi), mask=lane01)
            return (ptr, base)

        _, base_end = lax.fori_loop(
            0, jnp.maximum(pop_pos - 1, 0), fy_body,
            (jnp.int32(0), jnp.int32(0)))

        def vlookup(list_ref, i_ref, e_ref, r):
            w = jnp.zeros((16,), jnp.int32)
            for step in (8, 4, 2, 1):
                probe = jnp.minimum(w + (step - 1), _NSUB - 1)
                ipv = plsc.load_gather(i_ref, [probe])
                w = jnp.where(ipv <= r, w + step, w)
            w = jnp.minimum(w, _NSUB - 1)
            off = jnp.clip(r - plsc.load_gather(e_ref, [w]), 0, _CHUNK2 - 1)
            return plsc.load_gather(list_ref, [w, off])

        for c in range(2):
            r = perm_v[pl.ds(c * 16, 16)]
            pidx = vlookup(posl_v, ipos_v, epos_v, r)
            nidx = vlookup(negl_v, ineg_v, eneg_v, r - pop_pos)
            keep_v[pl.ds(c * 16, 16)] = jnp.where(r < pop_pos, pidx, nidx)

        bound_n = (pop_neg - 1).astype(jnp.uint32)

        def neg_body(k, carry):
            ptr, base = carry
            base = ensure(ptr, base)

            def do(p):
                return draw(p, base, bound_n)

            def skip(p):
                return jnp.uint32(0), p

            v, ptr = lax.cond(pop_neg > 1, do, skip, ptr)
            _swrite1(keep_v, 32 + k, v.astype(jnp.int32))
            return (ptr, base)

        lax.fori_loop(0, 96, neg_body, (jnp.int32(0), base_end))

        for c in range(2, 8):
            d = keep_v[pl.ds(c * 16, 16)]
            idx = vlookup(negl_v, ineg_v, eneg_v, d)
            keep_v[pl.ds(c * 16, 16)] = jnp.where(
                _splat(pop_neg) == 0, 0, idx)

        for c in range(8):
            kvec = keep_v[pl.ds(c * 16, 16)]
            avec = plsc.load_gather(amax_v, [kvec])
            lvec = plsc.load_gather(label_v, [avec])
            if c < 2:
                cls_v[pl.ds(c * 16, 16)] = lvec + 1
            else:
                cls_v[pl.ds(c * 16, 16)] = jnp.zeros((16,), jnp.int32)
            for col in range(4):
                g = plsc.load_gather(bbox_v, [avec, _splat(col)])
                mgt_t_v[col, pl.ds(c * 16, 16)] = g

        pltpu.async_copy(rois16_hbm.at[keep_v], roisg_v, sem).wait()
        for c in range(8):
            rows = c * 16 + iota
            for col in range(4):
                g = plsc.load_gather(roisg_v, [rows, _splat(col)])
                srois_t_v[col, pl.ds(c * 16, 16)] = g

        pltpu.sync_copy(cls_v, cls_hbm)
        pltpu.sync_copy(mgt_t_v, mgt_hbm)
        pltpu.sync_copy(srois_t_v, srois_hbm)


def _encode_tc_body(g_ref, p_ref, out_ref):
    g = g_ref[...]
    p = p_ref[...]
    gcx = (g[2:3] + g[0:1]) / 2.0
    gcy = (g[3:4] + g[1:2]) / 2.0
    gw = g[2:3] - g[0:1]
    gh = g[3:4] - g[1:2]
    pcx = (p[2:3] + p[0:1]) / 2.0
    pcy = (p[3:4] + p[1:2]) / 2.0
    pw = p[2:3] - p[0:1]
    ph = p[3:4] - p[1:2]
    tx = (gcx - pcx) / pw
    ty = (gcy - pcy) / ph
    tw = jnp.log(gw / pw)
    th = jnp.log(gh / ph)
    out_ref[...] = jnp.concatenate([tx, ty, tw, th], axis=0)


_encode_call = pl.pallas_call(
    _encode_tc_body,
    out_shape=jax.ShapeDtypeStruct((4, 128), jnp.float32),
)


def kernel(bbox, label, rois):
    bbox0 = bbox[0]
    label0 = label[0]
    rois_cat = jnp.concatenate([rois, bbox0], axis=0)
    rois_t = jnp.pad(rois_cat, ((0, _NPAD - _N), (0, 0))).T
    iou_max2, amax2 = _iou_call(rois_t, bbox0)
    iou_max = iou_max2.reshape(_NPAD)
    amax = amax2.reshape(_NPAD)
    rois16 = jnp.pad(rois_cat, ((0, 0), (0, 12)))
    stream = jnp.asarray(_STREAM_NP)
    cls, mgt_t, srois_t = _fused_call(
        iou_max, amax, stream, rois16, bbox0, label0)
    sample_rois = srois_t.T
    reg = _encode_call(mgt_t, srois_t).T
    return (cls, reg, sample_rois)
